# baseline (speedup 1.0000x reference)
import functools

import jax
import jax.numpy as jnp
from jax import lax
from jax.experimental import pallas as pl
from jax.experimental.pallas import tpu as pltpu
from jax.experimental.pallas import tpu_sc as plsc

MAX_HOUR = 73
LANES = 16


def _sc_argmax_kernel(n_tokens, num_workers):
    tok_per_tile = n_tokens // num_workers
    mesh = plsc.VectorSubcoreMesh(core_axis_name="c", subcore_axis_name="s")

    @functools.partial(
        pl.kernel,
        mesh=mesh,
        out_type=jax.ShapeDtypeStruct((n_tokens,), jnp.int32),
        scratch_types=[
            pltpu.VMEM((tok_per_tile, MAX_HOUR), jnp.float32),
            pltpu.VMEM((tok_per_tile,), jnp.int32),
        ],
        compiler_params=pltpu.CompilerParams(needs_layout_passes=False),
    )
    def k(oh_hbm, out_hbm, oh_v, idx_v):
        num_cores = jax.lax.axis_size("c")
        wid = lax.axis_index("s") * num_cores + lax.axis_index("c")
        base = wid * tok_per_tile
        pltpu.sync_copy(oh_hbm.at[pl.ds(base, tok_per_tile), :], oh_v)
        n_groups = tok_per_tile // LANES
        lane = jnp.arange(LANES, dtype=jnp.int32)

        def h_body(h, carry):
            hvec = jnp.broadcast_to(h, (LANES,))
            out = []
            for g in range(n_groups):
                vmax, vidx = carry[g]
                v = plsc.load_gather(oh_v, [lane + g * LANES, hvec])
                m = v > vmax
                out.append((jnp.where(m, v, vmax), jnp.where(m, h, vidx)))
            return tuple(out)

        init = tuple(
            (jnp.full((LANES,), -jnp.inf, jnp.float32),
             jnp.zeros((LANES,), jnp.int32))
            for _ in range(n_groups))
        final = lax.fori_loop(0, MAX_HOUR, h_body, init)
        for g in range(n_groups):
            idx_v[pl.ds(g * LANES, LANES)] = final[g][1]
        pltpu.sync_copy(idx_v, out_hbm.at[pl.ds(base, tok_per_tile)])

    return k


def _tc_body(hour_ref, x_ref, pe_ref, o_ref):
    blk = x_ref.shape[0]
    h_pad = pe_ref.shape[0]
    hour = hour_ref[0, 0, :].reshape(1, blk)
    hh = lax.broadcasted_iota(jnp.int32, (h_pad, blk), 0)
    onehot_t = (hh == hour).astype(jnp.float32)
    gathered = lax.dot_general(
        onehot_t, pe_ref[...], (((0,), (0,)), ((), ())),
        preferred_element_type=jnp.float32)
    o_ref[...] = x_ref[...] + gathered


def _tc_add(hour3, x2, pe_pad, blk):
    n, d = x2.shape
    grid = n // blk
    h_pad = pe_pad.shape[0]
    return pl.pallas_call(
        _tc_body,
        grid=(grid,),
        in_specs=[
            pl.BlockSpec((1, 1, blk), lambda i: (i, 0, 0)),
            pl.BlockSpec((blk, d), lambda i: (i, 0)),
            pl.BlockSpec((h_pad, d), lambda i: (0, 0)),
        ],
        out_specs=pl.BlockSpec((blk, d), lambda i: (i, 0)),
        out_shape=jax.ShapeDtypeStruct((n, d), jnp.float32),
    )(hour3, x2, pe_pad)


def kernel(x, hour_onehot, pe):
    b, l, d = x.shape
    n = b * l
    num_workers = 32
    blk = 512
    oh2 = hour_onehot.reshape(n, MAX_HOUR)
    hour = _sc_argmax_kernel(n, num_workers)(oh2)
    pe_pad = jnp.pad(pe[0], ((0, (-MAX_HOUR) % 8), (0, 0)))
    out = _tc_add(hour.reshape(n // blk, 1, blk), x.reshape(n, d), pe_pad, blk)
    return out.reshape(b, l, d)


# speedup vs baseline: 1.2040x; 1.2040x over previous
import functools

import jax
import jax.numpy as jnp
from jax import lax
from jax.experimental import pallas as pl
from jax.experimental.pallas import tpu as pltpu
from jax.experimental.pallas import tpu_sc as plsc

MAX_HOUR = 73
LANES = 16


def _sc_argmax_kernel(n_tokens, num_workers):
    tok_per_tile = n_tokens // num_workers
    mesh = plsc.VectorSubcoreMesh(core_axis_name="c", subcore_axis_name="s")

    @functools.partial(
        pl.kernel,
        mesh=mesh,
        out_type=jax.ShapeDtypeStruct((n_tokens,), jnp.int32),
        scratch_types=[
            pltpu.VMEM((MAX_HOUR, tok_per_tile), jnp.float32),
            pltpu.VMEM((tok_per_tile,), jnp.int32),
        ],
        compiler_params=pltpu.CompilerParams(needs_layout_passes=False),
    )
    def k(oh_hbm, out_hbm, oh_v, idx_v):
        num_cores = jax.lax.axis_size("c")
        wid = lax.axis_index("s") * num_cores + lax.axis_index("c")
        base = wid * tok_per_tile
        pltpu.sync_copy(oh_hbm.at[:, pl.ds(base, tok_per_tile)], oh_v)
        n_groups = tok_per_tile // LANES

        def h_body(h, carry):
            out = []
            for g in range(n_groups):
                vmax, vidx = carry[g]
                v = oh_v[h, pl.ds(g * LANES, LANES)]
                m = v > vmax
                out.append((jnp.where(m, v, vmax), jnp.where(m, h, vidx)))
            return tuple(out)

        init = tuple(
            (jnp.full((LANES,), -jnp.inf, jnp.float32),
             jnp.zeros((LANES,), jnp.int32))
            for _ in range(n_groups))
        final = lax.fori_loop(0, MAX_HOUR, h_body, init)
        for g in range(n_groups):
            idx_v[pl.ds(g * LANES, LANES)] = final[g][1]
        pltpu.sync_copy(idx_v, out_hbm.at[pl.ds(base, tok_per_tile)])

    return k


def _tc_body(hour_ref, x_ref, pe_ref, o_ref):
    blk = x_ref.shape[0]
    h_pad = pe_ref.shape[0]
    hour = hour_ref[0, 0, :].reshape(1, blk)
    hh = lax.broadcasted_iota(jnp.int32, (h_pad, blk), 0)
    onehot_t = (hh == hour).astype(jnp.float32)
    gathered = lax.dot_general(
        onehot_t, pe_ref[...], (((0,), (0,)), ((), ())),
        preferred_element_type=jnp.float32)
    o_ref[...] = x_ref[...] + gathered


def _tc_add(hour3, x2, pe_pad, blk):
    n, d = x2.shape
    grid = n // blk
    h_pad = pe_pad.shape[0]
    return pl.pallas_call(
        _tc_body,
        grid=(grid,),
        in_specs=[
            pl.BlockSpec((1, 1, blk), lambda i: (i, 0, 0)),
            pl.BlockSpec((blk, d), lambda i: (i, 0)),
            pl.BlockSpec((h_pad, d), lambda i: (0, 0)),
        ],
        out_specs=pl.BlockSpec((blk, d), lambda i: (i, 0)),
        out_shape=jax.ShapeDtypeStruct((n, d), jnp.float32),
    )(hour3, x2, pe_pad)


def kernel(x, hour_onehot, pe):
    b, l, d = x.shape
    n = b * l
    num_workers = 32
    blk = 512
    oh_t = jnp.transpose(hour_onehot, (2, 0, 1)).reshape(MAX_HOUR, n)
    hour = _sc_argmax_kernel(n, num_workers)(oh_t)
    pe_pad = jnp.pad(pe[0], ((0, (-MAX_HOUR) % 8), (0, 0)))
    out = _tc_add(hour.reshape(n // blk, 1, blk), x.reshape(n, d), pe_pad, blk)
    return out.reshape(b, l, d)


# speedup vs baseline: 1.2874x; 1.0692x over previous
import functools

import jax
import jax.numpy as jnp
from jax import lax
from jax.experimental import pallas as pl
from jax.experimental.pallas import tpu as pltpu
from jax.experimental.pallas import tpu_sc as plsc

MAX_HOUR = 73
LANES = 16


def _sc_argmax_kernel(n_tokens, num_workers):
    tok_per_tile = n_tokens // num_workers
    mesh = plsc.VectorSubcoreMesh(core_axis_name="c", subcore_axis_name="s")

    @functools.partial(
        pl.kernel,
        mesh=mesh,
        out_type=jax.ShapeDtypeStruct((n_tokens,), jnp.int32),
        scratch_types=[
            pltpu.VMEM((MAX_HOUR, tok_per_tile), jnp.float32),
            pltpu.VMEM((tok_per_tile,), jnp.int32),
        ],
        compiler_params=pltpu.CompilerParams(needs_layout_passes=False),
    )
    def k(oh_hbm, out_hbm, oh_v, idx_v):
        num_cores = jax.lax.axis_size("c")
        wid = lax.axis_index("s") * num_cores + lax.axis_index("c")
        base = wid * tok_per_tile
        pltpu.sync_copy(oh_hbm.at[:, pl.ds(base, tok_per_tile)], oh_v)
        n_groups = tok_per_tile // LANES

        def h_body(h, carry):
            out = []
            for g in range(n_groups):
                vmax, vidx = carry[g]
                v = oh_v[h, pl.ds(g * LANES, LANES)]
                m = v > vmax
                out.append((jnp.where(m, v, vmax), jnp.where(m, h, vidx)))
            return tuple(out)

        init = tuple(
            (jnp.full((LANES,), -jnp.inf, jnp.float32),
             jnp.zeros((LANES,), jnp.int32))
            for _ in range(n_groups))
        final = lax.fori_loop(0, MAX_HOUR, h_body, init)
        for g in range(n_groups):
            idx_v[pl.ds(g * LANES, LANES)] = final[g][1]
        pltpu.sync_copy(idx_v, out_hbm.at[pl.ds(base, tok_per_tile)])

    return k


def _tc_body(hour_ref, x_ref, pe_ref, o_ref):
    blk = x_ref.shape[0]
    h_pad = pe_ref.shape[0]
    hour = hour_ref[0, 0, :].reshape(1, blk)
    hh = lax.broadcasted_iota(jnp.int32, (h_pad, blk), 0)
    onehot_t = (hh == hour).astype(jnp.float32)
    gathered = lax.dot_general(
        onehot_t, pe_ref[...], (((0,), (0,)), ((), ())),
        preferred_element_type=jnp.float32)
    o_ref[...] = x_ref[...] + gathered


def _tc_add(hour3, x2, pe_pad, blk):
    n, d = x2.shape
    grid = n // blk
    h_pad = pe_pad.shape[0]
    return pl.pallas_call(
        _tc_body,
        grid=(grid,),
        in_specs=[
            pl.BlockSpec((1, 1, blk), lambda i: (i, 0, 0)),
            pl.BlockSpec((blk, d), lambda i: (i, 0)),
            pl.BlockSpec((h_pad, d), lambda i: (0, 0)),
        ],
        out_specs=pl.BlockSpec((blk, d), lambda i: (i, 0)),
        out_shape=jax.ShapeDtypeStruct((n, d), jnp.float32),
    )(hour3, x2, pe_pad)


def kernel(x, hour_onehot, pe):
    b, l, d = x.shape
    n = b * l
    num_workers = 32
    blk = 1024
    oh_t = jnp.transpose(hour_onehot, (2, 0, 1)).reshape(MAX_HOUR, n)
    hour = _sc_argmax_kernel(n, num_workers)(oh_t)
    pe_pad = jnp.pad(pe[0], ((0, (-MAX_HOUR) % 8), (0, 0)))
    out = _tc_add(hour.reshape(n // blk, 1, blk), x.reshape(n, d), pe_pad, blk)
    return out.reshape(b, l, d)


# speedup vs baseline: 1.3234x; 1.0280x over previous
import functools

import jax
import jax.numpy as jnp
from jax import lax
from jax.experimental import pallas as pl
from jax.experimental.pallas import tpu as pltpu
from jax.experimental.pallas import tpu_sc as plsc

MAX_HOUR = 73
LANES = 16


def _sc_argmax_kernel(n_tokens, num_workers):
    tok_per_tile = n_tokens // num_workers
    mesh = plsc.VectorSubcoreMesh(core_axis_name="c", subcore_axis_name="s")

    @functools.partial(
        pl.kernel,
        mesh=mesh,
        out_type=jax.ShapeDtypeStruct((n_tokens,), jnp.int32),
        scratch_types=[
            pltpu.VMEM((MAX_HOUR, tok_per_tile), jnp.float32),
            pltpu.VMEM((tok_per_tile,), jnp.int32),
        ],
        compiler_params=pltpu.CompilerParams(needs_layout_passes=False),
    )
    def k(oh_hbm, out_hbm, oh_v, idx_v):
        num_cores = jax.lax.axis_size("c")
        wid = lax.axis_index("s") * num_cores + lax.axis_index("c")
        base = wid * tok_per_tile
        pltpu.sync_copy(oh_hbm.at[:, pl.ds(base, tok_per_tile)], oh_v)
        n_groups = tok_per_tile // LANES

        def h_body(h, carry):
            out = []
            for g in range(n_groups):
                vmax, vidx = carry[g]
                v = oh_v[h, pl.ds(g * LANES, LANES)]
                m = v > vmax
                out.append((jnp.where(m, v, vmax), jnp.where(m, h, vidx)))
            return tuple(out)

        init = tuple(
            (jnp.full((LANES,), -jnp.inf, jnp.float32),
             jnp.zeros((LANES,), jnp.int32))
            for _ in range(n_groups))
        final = lax.fori_loop(0, MAX_HOUR, h_body, init)
        for g in range(n_groups):
            idx_v[pl.ds(g * LANES, LANES)] = final[g][1]
        pltpu.sync_copy(idx_v, out_hbm.at[pl.ds(base, tok_per_tile)])

    return k


def _tc_body(hour_ref, x_ref, pe_ref, o_ref):
    blk = x_ref.shape[0]
    h_pad = pe_ref.shape[0]
    hour = hour_ref[0, 0, :].reshape(1, blk)
    hh = lax.broadcasted_iota(jnp.int32, (h_pad, blk), 0)
    onehot_t = (hh == hour).astype(jnp.float32)
    gathered = lax.dot_general(
        onehot_t, pe_ref[...], (((0,), (0,)), ((), ())),
        preferred_element_type=jnp.float32)
    o_ref[...] = x_ref[...] + gathered


def _tc_add(hour3, x2, pe_pad, blk):
    n, d = x2.shape
    grid = n // blk
    h_pad = pe_pad.shape[0]
    return pl.pallas_call(
        _tc_body,
        grid=(grid,),
        in_specs=[
            pl.BlockSpec((1, 1, blk), lambda i: (i, 0, 0)),
            pl.BlockSpec((blk, d), lambda i: (i, 0)),
            pl.BlockSpec((h_pad, d), lambda i: (0, 0)),
        ],
        out_specs=pl.BlockSpec((blk, d), lambda i: (i, 0)),
        out_shape=jax.ShapeDtypeStruct((n, d), jnp.float32),
    )(hour3, x2, pe_pad)


def kernel(x, hour_onehot, pe):
    b, l, d = x.shape
    n = b * l
    num_workers = 32
    blk = 2048
    oh_t = jnp.transpose(hour_onehot, (2, 0, 1)).reshape(MAX_HOUR, n)
    hour = _sc_argmax_kernel(n, num_workers)(oh_t)
    pe_pad = jnp.pad(pe[0], ((0, (-MAX_HOUR) % 8), (0, 0)))
    out = _tc_add(hour.reshape(n // blk, 1, blk), x.reshape(n, d), pe_pad, blk)
    return out.reshape(b, l, d)


# speedup vs baseline: 1.3509x; 1.0208x over previous
import functools

import jax
import jax.numpy as jnp
from jax import lax
from jax.experimental import pallas as pl
from jax.experimental.pallas import tpu as pltpu
from jax.experimental.pallas import tpu_sc as plsc

MAX_HOUR = 73
LANES = 16


def _sc_argmax_kernel(n_tokens, num_workers):
    tok_per_tile = n_tokens // num_workers
    mesh = plsc.VectorSubcoreMesh(core_axis_name="c", subcore_axis_name="s")

    @functools.partial(
        pl.kernel,
        mesh=mesh,
        out_type=jax.ShapeDtypeStruct((n_tokens,), jnp.int32),
        scratch_types=[
            pltpu.VMEM((MAX_HOUR * tok_per_tile,), jnp.float32),
            pltpu.VMEM((tok_per_tile,), jnp.int32),
            pltpu.SemaphoreType.DMA,
        ],
        compiler_params=pltpu.CompilerParams(needs_layout_passes=False),
    )
    def k(oh_hbm, out_hbm, oh_v, idx_v, sem):
        num_cores = jax.lax.axis_size("c")
        wid = lax.axis_index("s") * num_cores + lax.axis_index("c")
        base = wid * tok_per_tile
        copies = [
            pltpu.make_async_copy(
                oh_hbm.at[pl.ds(h * n_tokens + base, tok_per_tile)],
                oh_v.at[pl.ds(h * tok_per_tile, tok_per_tile)], sem)
            for h in range(MAX_HOUR)
        ]
        for c in copies:
            c.start()
        for c in copies:
            c.wait()
        n_groups = tok_per_tile // LANES

        def h_body(h, carry):
            out = []
            for g in range(n_groups):
                vmax, vidx = carry[g]
                v = oh_v[pl.ds(h * tok_per_tile + g * LANES, LANES)]
                m = v > vmax
                out.append((jnp.where(m, v, vmax), jnp.where(m, h, vidx)))
            return tuple(out)

        init = tuple(
            (jnp.full((LANES,), -jnp.inf, jnp.float32),
             jnp.zeros((LANES,), jnp.int32))
            for _ in range(n_groups))
        final = lax.fori_loop(0, MAX_HOUR, h_body, init)
        for g in range(n_groups):
            idx_v[pl.ds(g * LANES, LANES)] = final[g][1]
        pltpu.sync_copy(idx_v, out_hbm.at[pl.ds(base, tok_per_tile)])

    return k


def _tc_body(hour_ref, x_ref, pe_ref, o_ref):
    blk = x_ref.shape[0]
    h_pad = pe_ref.shape[0]
    hour = hour_ref[0, 0, :].reshape(1, blk)
    hh = lax.broadcasted_iota(jnp.int32, (h_pad, blk), 0)
    onehot_t = (hh == hour).astype(jnp.float32)
    gathered = lax.dot_general(
        onehot_t, pe_ref[...], (((0,), (0,)), ((), ())),
        preferred_element_type=jnp.float32)
    o_ref[...] = x_ref[...] + gathered


def _tc_add(hour3, x2, pe_pad, blk):
    n, d = x2.shape
    grid = n // blk
    h_pad = pe_pad.shape[0]
    return pl.pallas_call(
        _tc_body,
        grid=(grid,),
        in_specs=[
            pl.BlockSpec((1, 1, blk), lambda i: (i, 0, 0)),
            pl.BlockSpec((blk, d), lambda i: (i, 0)),
            pl.BlockSpec((h_pad, d), lambda i: (0, 0)),
        ],
        out_specs=pl.BlockSpec((blk, d), lambda i: (i, 0)),
        out_shape=jax.ShapeDtypeStruct((n, d), jnp.float32),
    )(hour3, x2, pe_pad)


def kernel(x, hour_onehot, pe):
    b, l, d = x.shape
    n = b * l
    num_workers = 32
    blk = 2048
    oh_t = jnp.transpose(hour_onehot, (2, 0, 1)).reshape(MAX_HOUR * n)
    hour = _sc_argmax_kernel(n, num_workers)(oh_t)
    pe_pad = jnp.pad(pe[0], ((0, (-MAX_HOUR) % 8), (0, 0)))
    out = _tc_add(hour.reshape(n // blk, 1, blk), x.reshape(n, d), pe_pad, blk)
    return out.reshape(b, l, d)


# speedup vs baseline: 1.3717x; 1.0154x over previous
import functools

import jax
import jax.numpy as jnp
from jax import lax
from jax.experimental import pallas as pl
from jax.experimental.pallas import tpu as pltpu
from jax.experimental.pallas import tpu_sc as plsc

MAX_HOUR = 73
LANES = 16


def _sc_argmax_kernel(n_tokens, num_workers):
    tok_per_tile = n_tokens // num_workers
    mesh = plsc.VectorSubcoreMesh(core_axis_name="c", subcore_axis_name="s")

    @functools.partial(
        pl.kernel,
        mesh=mesh,
        out_type=jax.ShapeDtypeStruct((n_tokens,), jnp.int32),
        scratch_types=[
            pltpu.VMEM((MAX_HOUR * tok_per_tile,), jnp.float32),
            pltpu.VMEM((tok_per_tile,), jnp.int32),
            pltpu.SemaphoreType.DMA,
        ],
        compiler_params=pltpu.CompilerParams(needs_layout_passes=False),
    )
    def k(oh_hbm, out_hbm, oh_v, idx_v, sem):
        num_cores = jax.lax.axis_size("c")
        wid = lax.axis_index("s") * num_cores + lax.axis_index("c")
        base = wid * tok_per_tile
        copies = [
            pltpu.make_async_copy(
                oh_hbm.at[pl.ds(h * n_tokens + base, tok_per_tile)],
                oh_v.at[pl.ds(h * tok_per_tile, tok_per_tile)], sem)
            for h in range(MAX_HOUR)
        ]
        for c in copies:
            c.start()
        for c in copies:
            c.wait()
        n_groups = tok_per_tile // LANES

        def h_body(h, carry):
            out = []
            for g in range(n_groups):
                vmax, vidx = carry[g]
                v = oh_v[pl.ds(h * tok_per_tile + g * LANES, LANES)]
                m = v > vmax
                out.append((jnp.where(m, v, vmax), jnp.where(m, h, vidx)))
            return tuple(out)

        init = tuple(
            (jnp.full((LANES,), -jnp.inf, jnp.float32),
             jnp.zeros((LANES,), jnp.int32))
            for _ in range(n_groups))
        final = lax.fori_loop(0, MAX_HOUR, h_body, init)
        for g in range(n_groups):
            idx_v[pl.ds(g * LANES, LANES)] = final[g][1]
        tc = wid // 2
        b0 = 2 * (wid % 2)
        seq_len = n_tokens // 4
        run = tok_per_tile // 2
        pltpu.sync_copy(idx_v.at[pl.ds(0, run)],
                        out_hbm.at[pl.ds(b0 * seq_len + tc * run, run)])
        pltpu.sync_copy(idx_v.at[pl.ds(run, run)],
                        out_hbm.at[pl.ds((b0 + 1) * seq_len + tc * run, run)])

    return k


def _tc_body(hour_ref, x_ref, pe_ref, o_ref):
    blk = x_ref.shape[0]
    h_pad = pe_ref.shape[0]
    hour = hour_ref[0, 0, :].reshape(1, blk)
    hh = lax.broadcasted_iota(jnp.int32, (h_pad, blk), 0)
    onehot_t = (hh == hour).astype(jnp.float32)
    gathered = lax.dot_general(
        onehot_t, pe_ref[...], (((0,), (0,)), ((), ())),
        preferred_element_type=jnp.float32)
    o_ref[...] = x_ref[...] + gathered


def _tc_add(hour3, x2, pe_pad, blk):
    n, d = x2.shape
    grid = n // blk
    h_pad = pe_pad.shape[0]
    return pl.pallas_call(
        _tc_body,
        grid=(grid,),
        in_specs=[
            pl.BlockSpec((1, 1, blk), lambda i: (i, 0, 0)),
            pl.BlockSpec((blk, d), lambda i: (i, 0)),
            pl.BlockSpec((h_pad, d), lambda i: (0, 0)),
        ],
        out_specs=pl.BlockSpec((blk, d), lambda i: (i, 0)),
        out_shape=jax.ShapeDtypeStruct((n, d), jnp.float32),
    )(hour3, x2, pe_pad)


def kernel(x, hour_onehot, pe):
    b, l, d = x.shape
    n = b * l
    num_workers = 32
    blk = 2048
    oh_t = jnp.transpose(
        hour_onehot.reshape(b, l // 128, 128, MAX_HOUR),
        (3, 1, 0, 2)).reshape(MAX_HOUR * n)
    hour = _sc_argmax_kernel(n, num_workers)(oh_t)
    pe_pad = jnp.pad(pe[0], ((0, (-MAX_HOUR) % 8), (0, 0)))
    out = _tc_add(hour.reshape(n // blk, 1, blk), x.reshape(n, d), pe_pad, blk)
    return out.reshape(b, l, d)
